# Initial kernel scaffold; baseline (speedup 1.0000x reference)
import jax
import jax.numpy as jnp
from jax.experimental import pallas as pl


def kernel(l1_states, vertex_charges, l1_table, vertex_table):
    raise NotImplementedError("write your pallas kernel here")


# speedup vs baseline: 3.9434x; 3.9434x over previous
import functools

import jax
import jax.numpy as jnp
from jax import lax
from jax.experimental import pallas as pl
from jax.experimental.pallas import tpu as pltpu
from jax.experimental.pallas import tpu_sc as plsc

DIM = 256
L1W = DIM - 4
NB, SEQ = 4096, 50
B = NB * SEQ
NC, NS = 2, 16
NW = NC * NS
BPW = B // NW
CH = 128
NCHUNK = BPW // CH


def _prep_body(l1s_ref, vc_ref, l1t_ref, vt_ref, fused_ref, comb_ref):
    l1 = jnp.clip(l1s_ref[...].astype(jnp.int32), 0, 255)
    v = jnp.clip(vc_ref[...].astype(jnp.int32), 0, 3)
    fused_ref[...] = l1 * 4 + v
    t = l1t_ref[...]
    comb_ref[:, :L1W] = jnp.broadcast_to(t[:, None, :], (256, 4, L1W)).reshape(1024, L1W)
    vt = vt_ref[...]
    comb_ref[:, L1W:] = jnp.broadcast_to(vt[None, :, :], (256, 4, 4)).reshape(1024, 4)


_prep = pl.pallas_call(
    _prep_body,
    out_shape=[
        jax.ShapeDtypeStruct((NB, SEQ), jnp.int32),
        jax.ShapeDtypeStruct((1024, DIM), jnp.float32),
    ],
)


@functools.cache
def _make_sc_gather():
    @functools.partial(
        pl.kernel,
        out_type=jax.ShapeDtypeStruct((B, DIM), jnp.float32),
        mesh=plsc.VectorSubcoreMesh(core_axis_name="c", subcore_axis_name="s"),
        scratch_types=[
            pltpu.VMEM((NCHUNK, CH), jnp.int32),
            pltpu.VMEM((CH, DIM), jnp.float32),
            pltpu.SemaphoreType.DMA,
        ],
    )
    def _sc_gather(tbl_hbm, idx_hbm, out_hbm, idx_v, rows_v, sem):
        wid = lax.axis_index("s") * NC + lax.axis_index("c")
        pltpu.sync_copy(idx_hbm.at[wid], idx_v)

        def body(j, carry):
            pltpu.async_copy(tbl_hbm.at[idx_v.at[j]], rows_v, sem).wait()
            pltpu.sync_copy(rows_v, out_hbm.at[pl.ds(wid * BPW + j * CH, CH)])
            return carry

        lax.fori_loop(0, NCHUNK, body, 0)

    return _sc_gather


def kernel(l1_states, vertex_charges, l1_table, vertex_table):
    fused, comb = _prep(
        l1_states.astype(jnp.int32),
        vertex_charges.astype(jnp.int32),
        l1_table,
        vertex_table,
    )
    idx3 = fused.reshape(NW, NCHUNK, CH)
    out = _make_sc_gather()(comb, idx3)
    return out.reshape(NB, SEQ, DIM)


# speedup vs baseline: 4.0922x; 1.0377x over previous
import functools

import jax
import jax.numpy as jnp
from jax import lax
from jax.experimental import pallas as pl
from jax.experimental.pallas import tpu as pltpu
from jax.experimental.pallas import tpu_sc as plsc

DIM = 256
L1W = DIM - 4
NB, SEQ = 4096, 50
B = NB * SEQ
NC, NS = 2, 16
NW = NC * NS
BPW = B // NW
CH = 128
NCHUNK = BPW // CH


def _prep_body(l1s_ref, vc_ref, l1t_ref, vt_ref, fused_ref, comb_ref):
    l1 = jnp.clip(l1s_ref[...].astype(jnp.int32), 0, 255)
    v = jnp.clip(vc_ref[...].astype(jnp.int32), 0, 3)
    fused_ref[...] = l1 * 4 + v
    t = l1t_ref[...]
    comb_ref[:, :L1W] = jnp.broadcast_to(t[:, None, :], (256, 4, L1W)).reshape(1024, L1W)
    vt = vt_ref[...]
    comb_ref[:, L1W:] = jnp.broadcast_to(vt[None, :, :], (256, 4, 4)).reshape(1024, 4)


_prep = pl.pallas_call(
    _prep_body,
    out_shape=[
        jax.ShapeDtypeStruct((NB, SEQ), jnp.int32),
        jax.ShapeDtypeStruct((1024, DIM), jnp.float32),
    ],
)


@functools.cache
def _make_sc_gather():
    @functools.partial(
        pl.kernel,
        out_type=jax.ShapeDtypeStruct((B, DIM), jnp.float32),
        mesh=plsc.VectorSubcoreMesh(core_axis_name="c", subcore_axis_name="s"),
        scratch_types=[
            pltpu.VMEM((NCHUNK, CH), jnp.int32),
            pltpu.VMEM((CH, DIM), jnp.float32),
            pltpu.VMEM((CH, DIM), jnp.float32),
            pltpu.SemaphoreType.DMA,
            pltpu.SemaphoreType.DMA,
        ],
    )
    def _sc_gather(tbl_hbm, idx_hbm, out_hbm, idx_v, rows0, rows1, sem0, sem1):
        wid = lax.axis_index("s") * NC + lax.axis_index("c")
        base = wid * BPW
        pltpu.sync_copy(idx_hbm.at[wid], idx_v)
        rows = (rows0, rows1)
        sems = (sem0, sem1)

        pltpu.async_copy(tbl_hbm.at[idx_v.at[0]], rows0, sem0)

        def outer(jo, carry):
            for b in range(2):
                j = jo * 2 + b
                nb = 1 - b

                @pl.when(j + 1 < NCHUNK)
                def _():
                    pltpu.async_copy(tbl_hbm.at[idx_v.at[j + 1]], rows[nb], sems[nb])

                pltpu.make_async_copy(tbl_hbm.at[pl.ds(0, CH)], rows[b], sems[b]).wait()
                pltpu.sync_copy(rows[b], out_hbm.at[pl.ds(base + j * CH, CH)])
            return carry

        lax.fori_loop(0, NCHUNK // 2, outer, 0)

    return _sc_gather


def kernel(l1_states, vertex_charges, l1_table, vertex_table):
    fused, comb = _prep(
        l1_states.astype(jnp.int32),
        vertex_charges.astype(jnp.int32),
        l1_table,
        vertex_table,
    )
    idx3 = fused.reshape(NW, NCHUNK, CH)
    out = _make_sc_gather()(comb, idx3)
    return out.reshape(NB, SEQ, DIM)
